# baseline (speedup 1.0000x reference)
import functools

import jax
import jax.numpy as jnp
from jax import lax
from jax.experimental import pallas as pl
from jax.experimental.pallas import tpu as pltpu
from jax.experimental.pallas import tpu_sc as plsc

E = 8
D = 2048
F = 4096
A = 32
T = 256
N_TOK = 8192
P = N_TOK + E * T
NT = P // T
TC_T = 256
NC_T = N_TOK // TC_T

NW = 32
ROWS_A = P // NW
CH_A = 16
NCH_A = ROWS_A // CH_A
ROWS_B = N_TOK // NW

_SC_MESH = plsc.VectorSubcoreMesh(core_axis_name="c", subcore_axis_name="s")


def _worker_id():
    return lax.axis_index("s") * 2 + lax.axis_index("c")


@functools.partial(
    pl.kernel,
    mesh=_SC_MESH,
    out_type=jax.ShapeDtypeStruct((P, D), jnp.float32),
    scratch_types=[
        pltpu.VMEM((ROWS_A,), jnp.int32),
        pltpu.VMEM((CH_A, D), jnp.float32),
        pltpu.SemaphoreType.DMA,
    ],
)
def _sc_dispatch(x_hbm, tok_hbm, xs_hbm, idx_v, rows_v, sem):
    w = _worker_id()
    base = w * ROWS_A
    pltpu.sync_copy(tok_hbm.at[pl.ds(base, ROWS_A)], idx_v)

    def chunk(c, _):
        pltpu.async_copy(
            x_hbm.at[idx_v.at[pl.ds(c * CH_A, CH_A)]], rows_v, sem
        ).wait()
        pltpu.sync_copy(rows_v, xs_hbm.at[pl.ds(base + c * CH_A, CH_A)])
        return 0

    lax.fori_loop(0, NCH_A, chunk, 0)


@functools.partial(
    pl.kernel,
    mesh=_SC_MESH,
    out_type=jax.ShapeDtypeStruct((N_TOK, 128), jnp.float32),
    scratch_types=[
        pltpu.VMEM((ROWS_B,), jnp.int32),
        pltpu.VMEM((ROWS_B, 128), jnp.float32),
        pltpu.SemaphoreType.DMA,
    ],
)
def _sc_combine(ys_hbm, pos_hbm, out_hbm, idx_v, rows_v, sem):
    w = _worker_id()
    base = w * ROWS_B
    pltpu.sync_copy(pos_hbm.at[pl.ds(base, ROWS_B)], idx_v)
    pltpu.async_copy(ys_hbm.at[idx_v], rows_v, sem).wait()
    pltpu.sync_copy(rows_v, out_hbm.at[pl.ds(base, ROWS_B)])


def _expert_body(exp_ref, x_ref, w1_ref, b1_ref, w2_ref, b2_ref, o_ref):
    xb = x_ref[...].astype(jnp.bfloat16)
    h = jnp.dot(xb, w1_ref[0], preferred_element_type=jnp.float32)
    h = jnp.maximum(h + b1_ref[0], 0.0).astype(jnp.bfloat16)
    y = jnp.dot(h, w2_ref[0], preferred_element_type=jnp.float32)
    o_ref[:, 0:A] = y + b2_ref[0]


def _expert_matmul(expert_of_tile, xs, W1, b1, W2, b2):
    grid_spec = pltpu.PrefetchScalarGridSpec(
        num_scalar_prefetch=1,
        grid=(NT,),
        in_specs=[
            pl.BlockSpec((T, D), lambda t, exp: (t, 0)),
            pl.BlockSpec((1, D, F), lambda t, exp: (exp[t], 0, 0)),
            pl.BlockSpec((1, 1, F), lambda t, exp: (exp[t], 0, 0)),
            pl.BlockSpec((1, F, A), lambda t, exp: (exp[t], 0, 0)),
            pl.BlockSpec((1, 1, A), lambda t, exp: (exp[t], 0, 0)),
        ],
        out_specs=pl.BlockSpec((T, 128), lambda t, exp: (t, 0)),
    )
    return pl.pallas_call(
        _expert_body,
        grid_spec=grid_spec,
        out_shape=jax.ShapeDtypeStruct((P, 128), jnp.float32),
    )(expert_of_tile, xs, W1.astype(jnp.bfloat16), b1.reshape(E, 1, F),
      W2.astype(jnp.bfloat16), b2.reshape(E, 1, A))


def _critic_body(x_ref, wc1_ref, bc1_ref, wc2_ref, bc2_ref, o_ref):
    xb = x_ref[...].astype(jnp.bfloat16)
    h = jnp.dot(xb, wc1_ref[...], preferred_element_type=jnp.float32)
    h = jnp.maximum(h + bc1_ref[...], 0.0).astype(jnp.bfloat16)
    v = jnp.dot(h, wc2_ref[...], preferred_element_type=jnp.float32)
    o_ref[...] = v + bc2_ref[...]


def _critic(x, Wc1, bc1, Wc2, bc2):
    return pl.pallas_call(
        _critic_body,
        grid=(NC_T,),
        in_specs=[
            pl.BlockSpec((TC_T, D), lambda t: (t, 0)),
            pl.BlockSpec((D, F), lambda t: (0, 0)),
            pl.BlockSpec((1, F), lambda t: (0, 0)),
            pl.BlockSpec((F, 1), lambda t: (0, 0)),
            pl.BlockSpec((1, 1), lambda t: (0, 0)),
        ],
        out_specs=pl.BlockSpec((TC_T, 1), lambda t: (t, 0)),
        out_shape=jax.ShapeDtypeStruct((N_TOK, 1), jnp.float32),
    )(x, Wc1.astype(jnp.bfloat16), bc1.reshape(1, F),
      Wc2.astype(jnp.bfloat16), bc2.reshape(1, 1))


def kernel(obs, hete_pick, W1, b1, W2, b2, Wc1, bc1, Wc2, bc2):
    n_threads, n_agents, d = obs.shape
    x = obs.reshape(-1, d)
    pick = hete_pick.reshape(-1).astype(jnp.int32)

    onehot = (pick[:, None] == jnp.arange(E, dtype=jnp.int32)[None, :])
    counts = jnp.sum(onehot, axis=0, dtype=jnp.int32)
    padded = ((counts + T - 1) // T) * T
    starts = jnp.cumsum(padded) - padded
    ends = starts + padded
    rank = jnp.cumsum(onehot, axis=0, dtype=jnp.int32) - onehot
    pos = starts[pick] + jnp.take_along_axis(rank, pick[:, None], axis=1)[:, 0]
    token_id = jnp.zeros((P,), jnp.int32).at[pos].set(
        jnp.arange(N_TOK, dtype=jnp.int32))
    tile_starts = jnp.arange(NT, dtype=jnp.int32) * T
    expert_of_tile = jnp.minimum(
        jnp.searchsorted(ends, tile_starts, side="right"), E - 1
    ).astype(jnp.int32)

    xs = _sc_dispatch(x, token_id)
    ys = _expert_matmul(expert_of_tile, xs, W1, b1, W2, b2)
    logits = _sc_combine(ys, pos)[:, :A].reshape(n_threads, n_agents, A)
    value = _critic(x, Wc1, bc1, Wc2, bc2).reshape(n_threads, n_agents, 1)
    return logits, value
